# Initial kernel scaffold; baseline (speedup 1.0000x reference)
import jax
import jax.numpy as jnp
from jax.experimental import pallas as pl


def kernel(sent_vecs, concept_ids, node_type_ids, node_scores, adj_lengths, edge_index, edge_type, svec2nvec_w, svec2nvec_b, concept_emb, emb_node_type_w, emb_node_type_b, emb_score_w, emb_score_b, ee_w1, ee_b1, ee_w2, ee_b2, gnn_wk_x, gnn_wk_n, gnn_wk_e, gnn_bk, gnn_wm_x, gnn_wm_n, gnn_wm_e, gnn_bm, gnn_wq_x, gnn_wq_n, gnn_bq, gnn_mlp_w1, gnn_mlp_b1, gnn_mlp_w2, gnn_mlp_b2, Vh_w, Vh_b, Vx_w, Vx_b, pool_wq, pool_bq, pool_wk, pool_bk, pool_wv, pool_bv, sen_fc_w, sen_fc_b, gra_fc_w, gra_fc_b):
    raise NotImplementedError("write your pallas kernel here")


# speedup vs baseline: 1.0465x; 1.0465x over previous
import functools
import math

import jax
import jax.numpy as jnp
from jax.experimental import pallas as pl
from jax.experimental.pallas import tpu as pltpu

_BN_SCALE = 1.0 / math.sqrt(1.0 + 1e-5)
_BF16 = jnp.bfloat16
_PAR = pltpu.CompilerParams(dimension_semantics=("parallel",))
_SQRT_2_PI = math.sqrt(2.0 / math.pi)


def _gelu(x):
    return 0.5 * x * (1.0 + jnp.tanh(_SQRT_2_PI * (x + 0.044715 * x ** 3)))


def _ceil_to(x, m):
    return (x + m - 1) // m * m


def _pad_rows(x, rows):
    if x.shape[0] == rows:
        return x
    return jnp.pad(x, ((0, rows - x.shape[0]), (0, 0)))


def _row_tiling(m):
    tm = min(512, _ceil_to(m, 16))
    return tm, _ceil_to(m, tm)


def _dual_matmul_kernel(x_ref, n_ref, wx_ref, wn_ref, b_ref, o_ref, *, act):
    acc = jnp.dot(x_ref[...], wx_ref[...], preferred_element_type=jnp.float32)
    acc = acc + jnp.dot(n_ref[...], wn_ref[...], preferred_element_type=jnp.float32)
    acc = acc + b_ref[...]
    if act == "gelu":
        acc = _gelu(acc)
    o_ref[...] = acc


def _dual_matmul(x, n, wx, wn, b, act="none"):
    m = x.shape[0]
    cols = wx.shape[1]
    tm, mp = _row_tiling(m)
    out = pl.pallas_call(
        functools.partial(_dual_matmul_kernel, act=act),
        grid=(mp // tm,),
        in_specs=[
            pl.BlockSpec((tm, x.shape[1]), lambda i: (i, 0)),
            pl.BlockSpec((tm, n.shape[1]), lambda i: (i, 0)),
            pl.BlockSpec(wx.shape, lambda i: (0, 0)),
            pl.BlockSpec(wn.shape, lambda i: (0, 0)),
            pl.BlockSpec((1, cols), lambda i: (0, 0)),
        ],
        out_specs=pl.BlockSpec((tm, cols), lambda i: (i, 0)),
        out_shape=jax.ShapeDtypeStruct((mp, cols), jnp.float32),
        compiler_params=_PAR,
    )(_pad_rows(x, mp).astype(_BF16), _pad_rows(n, mp).astype(_BF16),
      wx.astype(_BF16), wn.astype(_BF16), b.reshape(1, cols).astype(jnp.float32))
    return out[:m]


def _linear_kernel(x_ref, w_ref, b_ref, o_ref, *, act):
    acc = jnp.dot(x_ref[...], w_ref[...], preferred_element_type=jnp.float32)
    acc = acc + b_ref[...]
    if act == "gelu":
        acc = _gelu(acc)
    o_ref[...] = acc


def _linear(x, w, b, act="none"):
    m = x.shape[0]
    cols = w.shape[1]
    tm, mp = _row_tiling(m)
    out = pl.pallas_call(
        functools.partial(_linear_kernel, act=act),
        grid=(mp // tm,),
        in_specs=[
            pl.BlockSpec((tm, x.shape[1]), lambda i: (i, 0)),
            pl.BlockSpec(w.shape, lambda i: (0, 0)),
            pl.BlockSpec((1, cols), lambda i: (0, 0)),
        ],
        out_specs=pl.BlockSpec((tm, cols), lambda i: (i, 0)),
        out_shape=jax.ShapeDtypeStruct((mp, cols), jnp.float32),
        compiler_params=_PAR,
    )(_pad_rows(x, mp).astype(_BF16), w.astype(_BF16),
      b.reshape(1, cols).astype(jnp.float32))
    return out[:m]


def _bnrelu_linear_kernel(h_ref, w_ref, b_ref, o_ref):
    h = jnp.maximum(h_ref[...] * _BN_SCALE, 0.0).astype(_BF16)
    o_ref[...] = jnp.dot(h, w_ref[...], preferred_element_type=jnp.float32) + b_ref[...]


def _bnrelu_linear(h, w, b):
    m = h.shape[0]
    cols = w.shape[1]
    tm, mp = _row_tiling(m)
    out = pl.pallas_call(
        _bnrelu_linear_kernel,
        grid=(mp // tm,),
        in_specs=[
            pl.BlockSpec((tm, h.shape[1]), lambda i: (i, 0)),
            pl.BlockSpec(w.shape, lambda i: (0, 0)),
            pl.BlockSpec((1, cols), lambda i: (0, 0)),
        ],
        out_specs=pl.BlockSpec((tm, cols), lambda i: (i, 0)),
        out_shape=jax.ShapeDtypeStruct((mp, cols), jnp.float32),
        compiler_params=_PAR,
    )(_pad_rows(h, mp), w.astype(_BF16), b.reshape(1, cols).astype(jnp.float32))
    return out[:m]


def _mlp2_kernel(x_ref, w1_ref, b1_ref, w2_ref, b2_ref, o_ref):
    h = jnp.dot(x_ref[...], w1_ref[...], preferred_element_type=jnp.float32)
    h = jnp.maximum((h + b1_ref[...]) * _BN_SCALE, 0.0).astype(_BF16)
    y = jnp.dot(h, w2_ref[...], preferred_element_type=jnp.float32) + b2_ref[...]
    o_ref[...] = _gelu(y)


def _mlp2_gelu(x, w1, b1, w2, b2):
    m = x.shape[0]
    hd = w1.shape[1]
    cols = w2.shape[1]
    tm, mp = _row_tiling(m)
    out = pl.pallas_call(
        _mlp2_kernel,
        grid=(mp // tm,),
        in_specs=[
            pl.BlockSpec((tm, x.shape[1]), lambda i: (i, 0)),
            pl.BlockSpec(w1.shape, lambda i: (0, 0)),
            pl.BlockSpec((1, hd), lambda i: (0, 0)),
            pl.BlockSpec(w2.shape, lambda i: (0, 0)),
            pl.BlockSpec((1, cols), lambda i: (0, 0)),
        ],
        out_specs=pl.BlockSpec((tm, cols), lambda i: (i, 0)),
        out_shape=jax.ShapeDtypeStruct((mp, cols), jnp.float32),
        compiler_params=_PAR,
    )(_pad_rows(x, mp).astype(_BF16), w1.astype(_BF16),
      b1.reshape(1, hd).astype(jnp.float32), w2.astype(_BF16),
      b2.reshape(1, cols).astype(jnp.float32))
    return out[:m]


def _edge_tables_kernel(t_ref, w_ref, b_ref, o_ref):
    o_ref[0] = (jnp.dot(t_ref[...], w_ref[0], preferred_element_type=jnp.float32)
                + b_ref[0])


def _edge_tables(edge_tab, w_stack, b_stack):
    k, dd, two_d = w_stack.shape
    u = edge_tab.shape[0]
    return pl.pallas_call(
        _edge_tables_kernel,
        grid=(k,),
        in_specs=[
            pl.BlockSpec((u, dd), lambda l: (0, 0)),
            pl.BlockSpec((1, dd, two_d), lambda l: (l, 0, 0)),
            pl.BlockSpec((1, 1, two_d), lambda l: (l, 0, 0)),
        ],
        out_specs=pl.BlockSpec((1, u, two_d), lambda l: (l, 0, 0)),
        out_shape=jax.ShapeDtypeStruct((k, u, two_d), jnp.float32),
        compiler_params=_PAR,
    )(edge_tab.astype(_BF16), w_stack.astype(_BF16), b_stack.astype(jnp.float32))


_D = 256
_N_NTYPE = 4
_N_ETYPE = 38
_K_LAYERS = 5
_HEADS = 4
_N_ATT = 2


def kernel(sent_vecs, concept_ids, node_type_ids, node_scores, adj_lengths,
           edge_index, edge_type,
           svec2nvec_w, svec2nvec_b, concept_emb,
           emb_node_type_w, emb_node_type_b, emb_score_w, emb_score_b,
           ee_w1, ee_b1, ee_w2, ee_b2,
           gnn_wk_x, gnn_wk_n, gnn_wk_e, gnn_bk,
           gnn_wm_x, gnn_wm_n, gnn_wm_e, gnn_bm,
           gnn_wq_x, gnn_wq_n, gnn_bq,
           gnn_mlp_w1, gnn_mlp_b1, gnn_mlp_w2, gnn_mlp_b2,
           Vh_w, Vh_b, Vx_w, Vx_b,
           pool_wq, pool_bq, pool_wk, pool_bk, pool_wv, pool_bv,
           sen_fc_w, sen_fc_b, gra_fc_w, gra_fc_b):
    bs, nc, sent_dim = sent_vecs.shape
    n_node = concept_ids.shape[2]
    d = _D
    heads = _HEADS
    dph = d // heads
    bq = bs * nc
    n_total = bq * n_node
    sv = sent_vecs.reshape(bq, sent_dim)
    ci = concept_ids.reshape(bq, n_node)
    nt = node_type_ids.reshape(bq, n_node)
    ns_in = node_scores.reshape(bq, n_node, 1)
    al = adj_lengths.reshape(bq)

    offs = (jnp.arange(bq, dtype=edge_index.dtype) * n_node)[:, None]
    e_src = (edge_index[:, 0, :] + offs).reshape(-1)
    e_tgt = (edge_index[:, 1, :] + offs).reshape(-1)
    et_flat = edge_type.reshape(-1)
    loop_index = jnp.arange(n_total, dtype=e_src.dtype)
    src = jnp.concatenate([e_src, loop_index])
    tgt = jnp.concatenate([e_tgt, loop_index])

    gnn_input0 = _gelu(sv @ svec2nvec_w + svec2nvec_b)[:, None, :]
    gnn_input1 = concept_emb[ci[:, 1:] - 1]
    h_in = jnp.concatenate([gnn_input0, gnn_input1], axis=1).reshape(n_total, d)

    _mask = (jnp.arange(n_node)[None, :] < al[:, None]).astype(jnp.float32)
    ns = -ns_in
    ns = ns - ns[:, 0:1, :]
    ns = ns[:, :, 0] * _mask
    mean_norm = jnp.abs(ns).sum(axis=1) / al.astype(jnp.float32)
    ns = ns / (mean_norm[:, None] + 1e-5)

    nt_flat = nt.reshape(-1)
    t_onehot = jax.nn.one_hot(nt_flat, _N_NTYPE, dtype=jnp.float32)
    type_emb = _linear(t_onehot, emb_node_type_w, emb_node_type_b, act="gelu")
    js = jnp.power(1.1, jnp.arange(d // 2, dtype=jnp.float32))[None, :]
    bsin = jnp.sin(js * ns.reshape(n_total, 1))
    score_emb = _linear(bsin, emb_score_w, emb_score_b, act="gelu")
    nfe = jnp.concatenate([type_emb, score_emb], axis=1)

    n_uid = (_N_ETYPE + 1) * _N_NTYPE * _N_NTYPE
    u = jnp.arange(n_uid)
    u_et, u_ht, u_tt = u // 16, (u % 16) // 4, u % 4
    w1b = ee_w1.astype(jnp.float32)
    h1_tab = (w1b[u_et] + w1b[_N_ETYPE + 1 + u_ht]
              + w1b[_N_ETYPE + 1 + _N_NTYPE + u_tt] + ee_b1)
    edge_tab = _bnrelu_linear(h1_tab, ee_w2, ee_b2)

    uid_real = et_flat * 16 + nt_flat[e_src] * 4 + nt_flat[e_tgt]
    uid_loop = _N_ETYPE * 16 + nt_flat * 5
    uid = jnp.concatenate([uid_real, uid_loop])

    inv_s = 1.0 / math.sqrt(dph)
    w_e_stack = jnp.concatenate([gnn_wk_e, gnn_wm_e], axis=2)
    b_e_stack = jnp.concatenate([gnn_bk, gnn_bm], axis=1)[:, None, :]
    km_tabs = _edge_tables(edge_tab, w_e_stack, b_e_stack)

    wx_cat = jnp.concatenate([gnn_wk_x, gnn_wm_x, gnn_wq_x * inv_s], axis=2)
    wn_cat = jnp.concatenate([gnn_wk_n, gnn_wm_n, gnn_wq_n * inv_s], axis=2)
    zeros_b = jnp.zeros_like(gnn_bq)
    b_cat = jnp.concatenate([zeros_b, zeros_b, gnn_bq * inv_s], axis=1)

    x = h_in
    for layer in range(_K_LAYERS):
        proj = _dual_matmul(x, nfe, wx_cat[layer], wn_cat[layer], b_cat[layer])
        km = km_tabs[layer][uid]
        key_e = proj[:, :d][tgt] + km[:, :d]
        src_mq = proj[:, d:][src]
        qry_e = src_mq[:, d:]
        scores = (qry_e * key_e).reshape(-1, heads, dph).sum(axis=2)
        smax = jax.ops.segment_max(scores, src, num_segments=n_total)
        ex = jnp.exp(scores - smax[src])
        ssum = jax.ops.segment_sum(ex, src, num_segments=n_total)
        alpha = ex / (ssum[src] + 1e-16)
        msg_e = src_mq[:, :d] + km[:, d:]
        weighted = (msg_e.reshape(-1, heads, dph) * alpha[:, :, None]).reshape(-1, d)
        aggr = jax.ops.segment_sum(weighted, tgt, num_segments=n_total)
        x = _mlp2_gelu(aggr, gnn_mlp_w1[layer], gnn_mlp_b1[layer],
                       gnn_mlp_w2[layer], gnn_mlp_b2[layer])

    gnn_out = _dual_matmul(h_in, x, Vh_w, Vx_w, Vh_b + Vx_b, act="gelu")
    gnn_out = gnn_out.reshape(bq, n_node, d)
    z_vecs = gnn_out[:, 0]

    mask = jnp.arange(n_node)[None, :] >= al[:, None]
    mask = mask | (nt == 3)
    all_masked = jnp.all(mask, axis=1)
    mask = mask.at[:, 0].set(mask[:, 0] & ~all_masked)

    dk = d // _N_ATT
    qs = (sv @ pool_wq + pool_bq).reshape(bq, _N_ATT, dk)
    wkv = jnp.concatenate([pool_wk, pool_wv], axis=1)
    bkv = jnp.concatenate([pool_bk, pool_bv], axis=0)
    kv = _linear(gnn_out.reshape(bq * n_node, d), wkv, bkv)
    ks = kv[:, :d].reshape(bq, n_node, _N_ATT, dk)
    vs = kv[:, d:].reshape(bq, n_node, _N_ATT, dk)
    attn = jnp.einsum("bhd,blhd->bhl", qs, ks) / math.sqrt(dk)
    attn = jnp.where(mask[:, None, :], -jnp.inf, attn)
    attn = jax.nn.softmax(attn, axis=-1)
    graph_vecs = jnp.einsum("bhl,blhd->bhd", attn, vs).reshape(bq, d)

    sen_sco = sv @ sen_fc_w + sen_fc_b
    gra_sco = jnp.concatenate([graph_vecs, z_vecs], axis=1) @ gra_fc_w + gra_fc_b
    logits = (sen_sco + gra_sco).reshape(bs, nc)
    return logits, [sen_sco.reshape(bs, nc), gra_sco.reshape(bs, nc)]


# speedup vs baseline: 5.0156x; 4.7929x over previous
import functools
import math

import jax
import jax.numpy as jnp
from jax.experimental import pallas as pl
from jax.experimental.pallas import tpu as pltpu

_BN_SCALE = 1.0 / math.sqrt(1.0 + 1e-5)
_BF16 = jnp.bfloat16
_PAR = pltpu.CompilerParams(dimension_semantics=("parallel",))
_SQRT_2_PI = math.sqrt(2.0 / math.pi)


def _gelu(x):
    return 0.5 * x * (1.0 + jnp.tanh(_SQRT_2_PI * (x + 0.044715 * x ** 3)))


def _ceil_to(x, m):
    return (x + m - 1) // m * m


def _pad_rows(x, rows):
    if x.shape[0] == rows:
        return x
    return jnp.pad(x, ((0, rows - x.shape[0]), (0, 0)))


def _row_tiling(m):
    tm = min(512, _ceil_to(m, 16))
    return tm, _ceil_to(m, tm)


def _linear_kernel(x_ref, w_ref, b_ref, o_ref, *, act):
    acc = jnp.dot(x_ref[...], w_ref[...], preferred_element_type=jnp.float32)
    acc = acc + b_ref[...]
    if act == "gelu":
        acc = _gelu(acc)
    o_ref[...] = acc


def _linear(x, w, b, act="none"):
    m = x.shape[0]
    cols = w.shape[1]
    tm, mp = _row_tiling(m)
    out = pl.pallas_call(
        functools.partial(_linear_kernel, act=act),
        grid=(mp // tm,),
        in_specs=[
            pl.BlockSpec((tm, x.shape[1]), lambda i: (i, 0)),
            pl.BlockSpec(w.shape, lambda i: (0, 0)),
            pl.BlockSpec((1, cols), lambda i: (0, 0)),
        ],
        out_specs=pl.BlockSpec((tm, cols), lambda i: (i, 0)),
        out_shape=jax.ShapeDtypeStruct((mp, cols), jnp.float32),
        compiler_params=_PAR,
    )(_pad_rows(x, mp).astype(_BF16), w.astype(_BF16),
      b.reshape(1, cols).astype(jnp.float32))
    return out[:m]


def _bnrelu_linear_kernel(h_ref, w_ref, b_ref, o_ref):
    h = jnp.maximum(h_ref[...] * _BN_SCALE, 0.0).astype(_BF16)
    o_ref[...] = jnp.dot(h, w_ref[...], preferred_element_type=jnp.float32) + b_ref[...]


def _bnrelu_linear(h, w, b):
    m = h.shape[0]
    cols = w.shape[1]
    tm, mp = _row_tiling(m)
    out = pl.pallas_call(
        _bnrelu_linear_kernel,
        grid=(mp // tm,),
        in_specs=[
            pl.BlockSpec((tm, h.shape[1]), lambda i: (i, 0)),
            pl.BlockSpec(w.shape, lambda i: (0, 0)),
            pl.BlockSpec((1, cols), lambda i: (0, 0)),
        ],
        out_specs=pl.BlockSpec((tm, cols), lambda i: (i, 0)),
        out_shape=jax.ShapeDtypeStruct((mp, cols), jnp.float32),
        compiler_params=_PAR,
    )(_pad_rows(h, mp), w.astype(_BF16), b.reshape(1, cols).astype(jnp.float32))
    return out[:m]


_TRANS_A = (((0,), (0,)), ((), ()))


def _gat_stack_kernel(hin_ref, nfe_ref, ee_ref, src_ref, tgt_ref,
                      wx_ref, wn_ref, we_ref, bcat_ref,
                      w1_ref, b1_ref, w2_ref, b2_ref,
                      vh_ref, vx_ref, vb_ref, o_ref,
                      *, n_layers, n_node, n_edge, d, heads):
    f32 = jnp.float32
    dph = d // heads
    src = src_ref[0]
    tgt = tgt_ref[0]
    iota_ne = jax.lax.broadcasted_iota(jnp.int32, (n_node, n_edge), 0)
    ohT_s = (iota_ne == src).astype(f32)
    ohT_t = (iota_ne == tgt).astype(f32)
    ch = jax.lax.broadcasted_iota(jnp.int32, (d, heads), 0) // dph
    hh = jax.lax.broadcasted_iota(jnp.int32, (d, heads), 1)
    hm = (ch == hh).astype(f32)
    chT = jax.lax.broadcasted_iota(jnp.int32, (heads, d), 1) // dph
    hhT = jax.lax.broadcasted_iota(jnp.int32, (heads, d), 0)
    hmT = (chT == hhT).astype(f32)

    hin = hin_ref[0]
    nfe = nfe_ref[0].astype(_BF16)
    ee = ee_ref[0].astype(_BF16)
    x = hin
    for l in range(n_layers):
        xb = x.astype(_BF16)
        proj = (jnp.dot(xb, wx_ref[l], preferred_element_type=f32)
                + jnp.dot(nfe, wn_ref[l], preferred_element_type=f32)
                + bcat_ref[l])
        km = jnp.dot(ee, we_ref[l], preferred_element_type=f32)
        key_n = proj[:, :d]
        mq_n = proj[:, d:]
        g_t = jax.lax.dot_general(ohT_t, key_n, _TRANS_A,
                                  preferred_element_type=f32)
        g_s = jax.lax.dot_general(ohT_s, mq_n, _TRANS_A,
                                  preferred_element_type=f32)
        key_e = g_t + km[:, :d]
        msg_e = g_s[:, :d] + km[:, d:]
        qry_e = g_s[:, d:]
        scores = jnp.dot(qry_e * key_e, hm, preferred_element_type=f32)
        gmax = jnp.max(scores, axis=0, keepdims=True)
        ex = jnp.exp(scores - gmax)
        ssum = jnp.dot(ohT_s, ex, preferred_element_type=f32)
        den = jax.lax.dot_general(ohT_s, ssum, _TRANS_A,
                                  preferred_element_type=f32)
        alpha = ex / (den + 1e-16)
        weighted = msg_e * jnp.dot(alpha, hmT, preferred_element_type=f32)
        aggr = jnp.dot(ohT_t, weighted, preferred_element_type=f32)
        h1 = jnp.maximum(
            (jnp.dot(aggr.astype(_BF16), w1_ref[l], preferred_element_type=f32)
             + b1_ref[l]) * _BN_SCALE, 0.0)
        x = _gelu(jnp.dot(h1.astype(_BF16), w2_ref[l], preferred_element_type=f32)
                  + b2_ref[l])
    out = _gelu(jnp.dot(hin.astype(_BF16), vh_ref[...], preferred_element_type=f32)
                + jnp.dot(x.astype(_BF16), vx_ref[...], preferred_element_type=f32)
                + vb_ref[...])
    o_ref[0] = out


def _gat_stack(hin3, nfe3, ee3, src3, tgt3, wx, wn, we, bcat,
               w1, b1, w2, b2, vh, vx, vb):
    g, n_node, d = hin3.shape
    n_edge = ee3.shape[1]
    n_layers = wx.shape[0]
    heads = 4
    fix2 = lambda s: pl.BlockSpec(s, lambda i: (0, 0))
    fix3 = lambda s: pl.BlockSpec(s, lambda i: (0, 0, 0))
    blk3 = lambda a, b: pl.BlockSpec((1, a, b), lambda i: (i, 0, 0))
    return pl.pallas_call(
        functools.partial(_gat_stack_kernel, n_layers=n_layers, n_node=n_node,
                          n_edge=n_edge, d=d, heads=heads),
        grid=(g,),
        in_specs=[
            blk3(n_node, d),
            blk3(n_node, d),
            blk3(n_edge, d),
            blk3(1, n_edge),
            blk3(1, n_edge),
            fix3(wx.shape),
            fix3(wn.shape),
            fix3(we.shape),
            fix3(bcat.shape),
            fix3(w1.shape),
            fix3(b1.shape),
            fix3(w2.shape),
            fix3(b2.shape),
            fix2(vh.shape),
            fix2(vx.shape),
            fix2(vb.shape),
        ],
        out_specs=blk3(n_node, d),
        out_shape=jax.ShapeDtypeStruct((g, n_node, d), jnp.float32),
        compiler_params=_PAR,
    )(hin3, nfe3, ee3, src3, tgt3, wx.astype(_BF16), wn.astype(_BF16),
      we.astype(_BF16), bcat.astype(jnp.float32),
      w1.astype(_BF16), b1.astype(jnp.float32),
      w2.astype(_BF16), b2.astype(jnp.float32),
      vh.astype(_BF16), vx.astype(_BF16), vb.astype(jnp.float32))


_D = 256
_N_NTYPE = 4
_N_ETYPE = 38
_HEADS = 4
_N_ATT = 2


def kernel(sent_vecs, concept_ids, node_type_ids, node_scores, adj_lengths,
           edge_index, edge_type,
           svec2nvec_w, svec2nvec_b, concept_emb,
           emb_node_type_w, emb_node_type_b, emb_score_w, emb_score_b,
           ee_w1, ee_b1, ee_w2, ee_b2,
           gnn_wk_x, gnn_wk_n, gnn_wk_e, gnn_bk,
           gnn_wm_x, gnn_wm_n, gnn_wm_e, gnn_bm,
           gnn_wq_x, gnn_wq_n, gnn_bq,
           gnn_mlp_w1, gnn_mlp_b1, gnn_mlp_w2, gnn_mlp_b2,
           Vh_w, Vh_b, Vx_w, Vx_b,
           pool_wq, pool_bq, pool_wk, pool_bk, pool_wv, pool_bv,
           sen_fc_w, sen_fc_b, gra_fc_w, gra_fc_b):
    bs, nc, sent_dim = sent_vecs.shape
    n_node = concept_ids.shape[2]
    d = _D
    dph = d // _HEADS
    bq = bs * nc
    n_total = bq * n_node
    e_per = edge_index.shape[2]
    n_edge = e_per + n_node

    sv = sent_vecs.reshape(bq, sent_dim)
    ci = concept_ids.reshape(bq, n_node)
    nt = node_type_ids.reshape(bq, n_node)
    ns_in = node_scores.reshape(bq, n_node, 1)
    al = adj_lengths.reshape(bq)

    loops = jnp.broadcast_to(jnp.arange(n_node, dtype=edge_index.dtype)[None, :],
                             (bq, n_node))
    src_l = jnp.concatenate([edge_index[:, 0, :], loops], axis=1)
    tgt_l = jnp.concatenate([edge_index[:, 1, :], loops], axis=1)
    src3 = src_l.reshape(bq, 1, n_edge).astype(jnp.int32)
    tgt3 = tgt_l.reshape(bq, 1, n_edge).astype(jnp.int32)

    gnn_input0 = _gelu(sv @ svec2nvec_w + svec2nvec_b)[:, None, :]
    gnn_input1 = concept_emb[ci[:, 1:] - 1]
    hin3 = jnp.concatenate([gnn_input0, gnn_input1], axis=1)

    _mask = (jnp.arange(n_node)[None, :] < al[:, None]).astype(jnp.float32)
    ns = -ns_in
    ns = ns - ns[:, 0:1, :]
    ns = ns[:, :, 0] * _mask
    mean_norm = jnp.abs(ns).sum(axis=1) / al.astype(jnp.float32)
    ns = ns / (mean_norm[:, None] + 1e-5)

    nt_flat = nt.reshape(-1)
    type_tab = _gelu(emb_node_type_w + emb_node_type_b[None, :])
    type_emb = type_tab[nt_flat]
    js = jnp.power(1.1, jnp.arange(d // 2, dtype=jnp.float32))[None, :]
    bsin = jnp.sin(js * ns.reshape(n_total, 1))
    score_emb = _linear(bsin, emb_score_w, emb_score_b, act="gelu")
    nfe3 = jnp.concatenate([type_emb, score_emb], axis=1).reshape(bq, n_node, d)

    n_uid = (_N_ETYPE + 1) * _N_NTYPE * _N_NTYPE
    u = jnp.arange(n_uid)
    u_et, u_ht, u_tt = u // 16, (u % 16) // 4, u % 4
    h1_tab = (ee_w1[u_et] + ee_w1[_N_ETYPE + 1 + u_ht]
              + ee_w1[_N_ETYPE + 1 + _N_NTYPE + u_tt] + ee_b1)
    edge_tab = _bnrelu_linear(h1_tab, ee_w2, ee_b2)

    nt_src = jnp.take_along_axis(nt, src_l[:, :e_per], axis=1)
    nt_tgt = jnp.take_along_axis(nt, tgt_l[:, :e_per], axis=1)
    uid_real = edge_type * 16 + nt_src * 4 + nt_tgt
    uid_loop = _N_ETYPE * 16 + nt * 5
    uid = jnp.concatenate([uid_real, uid_loop], axis=1)
    ee3 = edge_tab[uid]

    inv_s = 1.0 / math.sqrt(dph)
    wx_cat = jnp.concatenate([gnn_wk_x, gnn_wm_x, gnn_wq_x * inv_s], axis=2)
    wn_cat = jnp.concatenate([gnn_wk_n, gnn_wm_n, gnn_wq_n * inv_s], axis=2)
    b_cat = jnp.concatenate([gnn_bk, gnn_bm, gnn_bq * inv_s], axis=1)[:, None, :]
    we_cat = jnp.concatenate([gnn_wk_e, gnn_wm_e], axis=2)

    gnn_out = _gat_stack(hin3, nfe3, ee3, src3, tgt3,
                         wx_cat, wn_cat, we_cat, b_cat,
                         gnn_mlp_w1, gnn_mlp_b1[:, None, :],
                         gnn_mlp_w2, gnn_mlp_b2[:, None, :],
                         Vh_w, Vx_w, (Vh_b + Vx_b).reshape(1, d))
    z_vecs = gnn_out[:, 0]

    mask = jnp.arange(n_node)[None, :] >= al[:, None]
    mask = mask | (nt == 3)
    all_masked = jnp.all(mask, axis=1)
    mask = mask.at[:, 0].set(mask[:, 0] & ~all_masked)

    dk = d // _N_ATT
    qs = (sv @ pool_wq + pool_bq).reshape(bq, _N_ATT, dk)
    wkv = jnp.concatenate([pool_wk, pool_wv], axis=1)
    bkv = jnp.concatenate([pool_bk, pool_bv], axis=0)
    kv = _linear(gnn_out.reshape(bq * n_node, d), wkv, bkv)
    ks = kv[:, :d].reshape(bq, n_node, _N_ATT, dk)
    vs = kv[:, d:].reshape(bq, n_node, _N_ATT, dk)
    attn = jnp.einsum("bhd,blhd->bhl", qs, ks) / math.sqrt(dk)
    attn = jnp.where(mask[:, None, :], -jnp.inf, attn)
    attn = jax.nn.softmax(attn, axis=-1)
    graph_vecs = jnp.einsum("bhl,blhd->bhd", attn, vs).reshape(bq, d)

    sen_sco = sv @ sen_fc_w + sen_fc_b
    gra_sco = jnp.concatenate([graph_vecs, z_vecs], axis=1) @ gra_fc_w + gra_fc_b
    logits = (sen_sco + gra_sco).reshape(bs, nc)
    return logits, [sen_sco.reshape(bs, nc), gra_sco.reshape(bs, nc)]
